# baseline (speedup 1.0000x reference)
import functools

import jax
import jax.numpy as jnp
from jax import lax
from jax.experimental import pallas as pl
from jax.experimental.pallas import tpu as pltpu
from jax.experimental.pallas import tpu_sc as plsc

K = 4
R = 2048
D = 4096
NC = 2
NS = 16
NW = NC * NS
ROWS_PER_W = R // NW
NBUF = 8


def _make_sc_upsample():
    mesh = plsc.VectorSubcoreMesh(core_axis_name="c", subcore_axis_name="s")

    @functools.partial(
        pl.kernel,
        mesh=mesh,
        out_type=jax.ShapeDtypeStruct((R * K, D), jnp.float32),
        scratch_types=[
            pltpu.VMEM((NBUF, D), jnp.float32),
            pltpu.SemaphoreType.DMA,
            pltpu.SemaphoreType.DMA,
        ],
    )
    def upsample(xs_hbm, out_hbm, buf, lsem, ssem):
        wid = lax.axis_index("s") * NC + lax.axis_index("c")
        base = wid * ROWS_PER_W

        def batch(g, carry):
            row0 = base + g * NBUF
            loads = []
            for b in range(NBUF):
                loads.append(
                    pltpu.async_copy(
                        xs_hbm.at[pl.ds(row0 + b, 1)], buf.at[pl.ds(b, 1)], lsem
                    )
                )
            stores = []
            for b in range(NBUF):
                loads[b].wait()
                for r in range(K):
                    stores.append(
                        pltpu.async_copy(
                            buf.at[pl.ds(b, 1)],
                            out_hbm.at[pl.ds((row0 + b) * K + r, 1)],
                            ssem,
                        )
                    )
            for st in stores:
                st.wait()
            return carry

        lax.fori_loop(0, ROWS_PER_W // NBUF, batch, 0)

    return upsample


_sc_upsample = _make_sc_upsample()


def kernel(x, x_short):
    xs = x_short.reshape(R, D)
    out = _sc_upsample(xs)
    return out.reshape(R * K, 4, 1024)
